# baseline (speedup 1.0000x reference)
import jax
import jax.numpy as jnp
from jax.experimental import pallas as pl
from jax.experimental.pallas import tpu as pltpu

_EMBED_DIM = 1024
_MLP_DIM = 1024
_MHA_DIM = 128
_NEURONS = 4096
_HEADS = 16


def _combine_kernel(w2m_ref, w1m_ref, w2h_ref, w1h_ref, wc_ref, wh_ref):
    wc_ref[...] = jax.lax.dot_general(
        w2m_ref[...].astype(jnp.bfloat16), w1m_ref[...].astype(jnp.bfloat16),
        (((1,), (0,)), ((), ())),
        preferred_element_type=jnp.float32).astype(jnp.bfloat16)
    wh_ref[...] = jax.lax.dot_general(
        w2h_ref[...].astype(jnp.bfloat16), w1h_ref[...].astype(jnp.bfloat16),
        (((1,), (0,)), ((), ())),
        preferred_element_type=jnp.float32).astype(jnp.bfloat16)


def _router_kernel(x_ref, wc_ref, wh_ref, neurons_ref, heads_ref):
    x = x_ref[...].astype(jnp.bfloat16)
    neurons_ref[...] = jax.lax.dot_general(
        x, wc_ref[...], (((1,), (1,)), ((), ())),
        preferred_element_type=jnp.float32)
    heads_ref[...] = jax.lax.dot_general(
        x, wh_ref[...], (((1,), (1,)), ((), ())),
        preferred_element_type=jnp.float32)


def kernel(x, W1, ln_gamma, ln_beta, W2_mlp, W2_mha):
    del ln_gamma, ln_beta
    n_tok = x.shape[0]

    nb = 1024
    wc, wh = pl.pallas_call(
        _combine_kernel,
        grid=(_NEURONS // nb,),
        in_specs=[
            pl.BlockSpec((nb, _MLP_DIM), lambda j: (j, 0)),
            pl.BlockSpec((_MLP_DIM, _EMBED_DIM), lambda j: (0, 0)),
            pl.BlockSpec((_HEADS, _MHA_DIM), lambda j: (0, 0)),
            pl.BlockSpec((_MHA_DIM, _EMBED_DIM),
                         lambda j: (_MLP_DIM // _MHA_DIM, 0)),
        ],
        out_specs=[
            pl.BlockSpec((nb, _EMBED_DIM), lambda j: (j, 0)),
            pl.BlockSpec((_HEADS, _EMBED_DIM), lambda j: (0, 0)),
        ],
        out_shape=[
            jax.ShapeDtypeStruct((_NEURONS, _EMBED_DIM), jnp.bfloat16),
            jax.ShapeDtypeStruct((_HEADS, _EMBED_DIM), jnp.bfloat16),
        ],
        compiler_params=pltpu.CompilerParams(
            dimension_semantics=("arbitrary",)),
    )(W2_mlp, W1, W2_mha, W1)

    bt = 1024
    neurons, heads = pl.pallas_call(
        _router_kernel,
        grid=(n_tok // bt,),
        in_specs=[
            pl.BlockSpec((bt, _EMBED_DIM), lambda i: (i, 0)),
            pl.BlockSpec((_NEURONS, _EMBED_DIM), lambda i: (0, 0)),
            pl.BlockSpec((_HEADS, _EMBED_DIM), lambda i: (0, 0)),
        ],
        out_specs=[
            pl.BlockSpec((bt, _NEURONS), lambda i: (i, 0)),
            pl.BlockSpec((bt, _HEADS), lambda i: (i, 0)),
        ],
        out_shape=[
            jax.ShapeDtypeStruct((n_tok, _NEURONS), jnp.float32),
            jax.ShapeDtypeStruct((n_tok, _HEADS), jnp.float32),
        ],
        compiler_params=pltpu.CompilerParams(
            dimension_semantics=("arbitrary",)),
    )(x, wc, wh)
    return (neurons, heads)


# speedup vs baseline: 1.0342x; 1.0342x over previous
import jax
import jax.numpy as jnp
from jax.experimental import pallas as pl
from jax.experimental.pallas import tpu as pltpu

_EMBED_DIM = 1024
_MLP_DIM = 1024
_MHA_DIM = 128
_NEURONS = 4096
_HEADS = 16


def _fused_kernel(x_ref, w1_ref, w2m_ref, w2h_ref,
                  neurons_ref, heads_ref, wct, wht):
    i = pl.program_id(0)

    @pl.when(i == 0)
    def _():
        w1m = w1_ref[: _MLP_DIM, :].astype(jnp.bfloat16)
        wct[...] = jax.lax.dot_general(
            w1m, w2m_ref[...].astype(jnp.bfloat16),
            (((0,), (1,)), ((), ())),
            preferred_element_type=jnp.float32).astype(jnp.bfloat16)
        w1h = w1_ref[_MLP_DIM:, :].astype(jnp.bfloat16)
        wht[...] = jax.lax.dot_general(
            w1h, w2h_ref[...].astype(jnp.bfloat16),
            (((0,), (1,)), ((), ())),
            preferred_element_type=jnp.float32).astype(jnp.bfloat16)

    @pl.when(i > 0)
    def _():
        x = x_ref[...].astype(jnp.bfloat16)
        neurons_ref[...] = jax.lax.dot_general(
            x, wct[...], (((1,), (0,)), ((), ())),
            preferred_element_type=jnp.float32)
        heads_ref[...] = jax.lax.dot_general(
            x, wht[...], (((1,), (0,)), ((), ())),
            preferred_element_type=jnp.float32)


def kernel(x, W1, ln_gamma, ln_beta, W2_mlp, W2_mha):
    del ln_gamma, ln_beta
    n_tok = x.shape[0]
    bt = 512
    n = n_tok // bt
    neurons, heads = pl.pallas_call(
        _fused_kernel,
        grid=(n + 1,),
        in_specs=[
            pl.BlockSpec((bt, _EMBED_DIM),
                         lambda i: (jnp.maximum(i - 1, 0), 0)),
            pl.BlockSpec((_MLP_DIM + _MHA_DIM, _EMBED_DIM), lambda i: (0, 0)),
            pl.BlockSpec((_NEURONS, _MLP_DIM), lambda i: (0, 0)),
            pl.BlockSpec((_HEADS, _MHA_DIM), lambda i: (0, 0)),
        ],
        out_specs=[
            pl.BlockSpec((bt, _NEURONS),
                         lambda i: (jnp.maximum(i - 1, 0), 0)),
            pl.BlockSpec((bt, _HEADS),
                         lambda i: (jnp.maximum(i - 1, 0), 0)),
        ],
        out_shape=[
            jax.ShapeDtypeStruct((n_tok, _NEURONS), jnp.float32),
            jax.ShapeDtypeStruct((n_tok, _HEADS), jnp.float32),
        ],
        scratch_shapes=[
            pltpu.VMEM((_EMBED_DIM, _NEURONS), jnp.bfloat16),
            pltpu.VMEM((_EMBED_DIM, _HEADS), jnp.bfloat16),
        ],
        compiler_params=pltpu.CompilerParams(
            dimension_semantics=("arbitrary",)),
    )(x, W1, W2_mlp, W2_mha)
    return (neurons, heads)
